# baseline (speedup 1.0000x reference)
import functools

import jax
import jax.numpy as jnp
from jax import lax
from jax.experimental import pallas as pl
from jax.experimental.pallas import tpu as pltpu
from jax.experimental.pallas import tpu_sc as plsc

_B = 16
_SEG = 2048
_D = 1024
_IMAX = 64
_HALF = _D // 2
_LANES = 16

_ROWS = 2048
_C = _SEG // _ROWS


def _sc_instr_sums(hidden_states, instruction_lens):
    mesh = plsc.VectorSubcoreMesh(core_axis_name="c", subcore_axis_name="s")

    @functools.partial(
        pl.kernel,
        out_type=jax.ShapeDtypeStruct((_B, _D), jnp.float32),
        mesh=mesh,
        compiler_params=pltpu.CompilerParams(needs_layout_passes=False),
        scratch_types=[
            pltpu.VMEM((_IMAX, _HALF), jnp.float32),
            pltpu.VMEM((_HALF,), jnp.float32),
            pltpu.VMEM((_B,), jnp.int32),
        ],
    )
    def k(hs_hbm, ilen_hbm, out_hbm, buf, acc, ilen_vmem):
        wid = lax.axis_index("s") * 2 + lax.axis_index("c")
        seg = wid // 2
        col0 = (wid % 2) * _HALF
        pltpu.sync_copy(ilen_hbm, ilen_vmem)
        pltpu.sync_copy(
            hs_hbm.at[pl.ds(seg * _SEG, _IMAX), pl.ds(col0, _HALF)], buf
        )
        nvec = plsc.load_gather(ilen_vmem, [jnp.full((_LANES,), seg, jnp.int32)])
        zero = jnp.zeros((_LANES,), jnp.float32)
        for g in range(_HALF // _LANES):
            sl = pl.ds(g * _LANES, _LANES)

            def body(r, reg):
                rvec = lax.broadcast(r, (_LANES,))
                return reg + jnp.where(rvec < nvec, buf[r, sl], zero)

            acc[sl] = lax.fori_loop(0, _IMAX - 1, body, zero)
        pltpu.sync_copy(acc, out_hbm.at[seg, pl.ds(col0, _HALF)])

    return k(hidden_states, instruction_lens)


_SEGS_PER_STEP = 2


def _tc_body(plen_ref, ilen_ref, hs_ref, instr_ref, out_ref):
    pid = pl.program_id(0)
    for j in range(_SEGS_PER_STEP):
        b = pid * _SEGS_PER_STEP + j
        dense = jnp.sum(hs_ref[j * _SEG : (j + 1) * _SEG], axis=0, keepdims=True)
        denom = (plen_ref[b] - ilen_ref[b]).astype(jnp.float32)
        mean = (dense - instr_ref[j]) / denom
        norm = jnp.maximum(jnp.sqrt(jnp.sum(mean * mean)), 1e-12)
        out_ref[j] = mean / norm


def kernel(hidden_states, prompt_lens, instruction_lens):
    instr_sums = _sc_instr_sums(hidden_states, instruction_lens)
    out = pl.pallas_call(
        _tc_body,
        grid=(_B // _SEGS_PER_STEP,),
        in_specs=[
            pl.BlockSpec(memory_space=pltpu.SMEM),
            pl.BlockSpec(memory_space=pltpu.SMEM),
            pl.BlockSpec((_SEGS_PER_STEP * _SEG, _D), lambda b: (b, 0)),
            pl.BlockSpec((_SEGS_PER_STEP, 1, _D), lambda b: (b, 0, 0)),
        ],
        out_specs=pl.BlockSpec((_SEGS_PER_STEP, 1, _D), lambda b: (b, 0, 0)),
        out_shape=jax.ShapeDtypeStruct((_B, 1, _D), jnp.float32),
        compiler_params=pltpu.CompilerParams(dimension_semantics=("parallel",)),
    )(prompt_lens, instruction_lens, hidden_states, instr_sums.reshape(_B, 1, _D))
    return out.reshape(_B, _D)


# speedup vs baseline: 1.2196x; 1.2196x over previous
import functools

import jax
import jax.numpy as jnp
from jax import lax
from jax.experimental import pallas as pl
from jax.experimental.pallas import tpu as pltpu
from jax.experimental.pallas import tpu_sc as plsc

_B = 16
_SEG = 2048
_D = 1024
_IMAX = 64
_HALF = _D // 2
_LANES = 16

_ROWS = 2048
_C = _SEG // _ROWS


def _sc_instr_sums(hidden_states, instruction_lens):
    mesh = plsc.VectorSubcoreMesh(core_axis_name="c", subcore_axis_name="s")

    @functools.partial(
        pl.kernel,
        out_type=jax.ShapeDtypeStruct((_B, _D), jnp.float32),
        mesh=mesh,
        compiler_params=pltpu.CompilerParams(needs_layout_passes=False),
        scratch_types=[
            pltpu.VMEM((_IMAX, _HALF), jnp.float32),
            pltpu.VMEM((_HALF,), jnp.float32),
            pltpu.VMEM((_B,), jnp.int32),
        ],
    )
    def k(hs_hbm, ilen_hbm, out_hbm, buf, acc, ilen_vmem):
        wid = lax.axis_index("s") * 2 + lax.axis_index("c")
        seg = wid // 2
        col0 = (wid % 2) * _HALF
        pltpu.sync_copy(ilen_hbm, ilen_vmem)
        pltpu.sync_copy(
            hs_hbm.at[pl.ds(seg * _SEG, _IMAX), pl.ds(col0, _HALF)], buf
        )
        nvec = plsc.load_gather(ilen_vmem, [jnp.full((_LANES,), seg, jnp.int32)])
        zero = jnp.zeros((_LANES,), jnp.float32)
        for g in range(_HALF // _LANES):
            sl = pl.ds(g * _LANES, _LANES)

            def body(r, reg):
                rvec = lax.broadcast(r, (_LANES,))
                return reg + jnp.where(rvec < nvec, buf[r, sl], zero)

            acc[sl] = lax.fori_loop(0, _IMAX - 1, body, zero)
        pltpu.sync_copy(acc, out_hbm.at[seg, pl.ds(col0, _HALF)])

    return k(hidden_states, instruction_lens)


_SEGS_PER_STEP = 2


def _tc_dense_body(hs_ref, out_ref):
    for j in range(_SEGS_PER_STEP):
        out_ref[j] = jnp.sum(hs_ref[j * _SEG : (j + 1) * _SEG], axis=0, keepdims=True)


def _tc_combine_body(plen_ref, ilen_ref, dense_ref, instr_ref, out_ref):
    for b in range(_B):
        denom = (plen_ref[b] - ilen_ref[b]).astype(jnp.float32)
        mean = (dense_ref[b] - instr_ref[b]) / denom
        norm = jnp.maximum(jnp.sqrt(jnp.sum(mean * mean)), 1e-12)
        out_ref[b] = mean / norm


def kernel(hidden_states, prompt_lens, instruction_lens):
    instr_sums = _sc_instr_sums(hidden_states, instruction_lens)
    dense_sums = pl.pallas_call(
        _tc_dense_body,
        grid=(_B // _SEGS_PER_STEP,),
        in_specs=[
            pl.BlockSpec((_SEGS_PER_STEP * _SEG, _D), lambda b: (b, 0)),
        ],
        out_specs=pl.BlockSpec((_SEGS_PER_STEP, 1, _D), lambda b: (b, 0, 0)),
        out_shape=jax.ShapeDtypeStruct((_B, 1, _D), jnp.float32),
        compiler_params=pltpu.CompilerParams(dimension_semantics=("parallel",)),
    )(hidden_states)
    out = pl.pallas_call(
        _tc_combine_body,
        in_specs=[
            pl.BlockSpec(memory_space=pltpu.SMEM),
            pl.BlockSpec(memory_space=pltpu.SMEM),
            pl.BlockSpec((_B, 1, _D), lambda: (0, 0, 0)),
            pl.BlockSpec((_B, 1, _D), lambda: (0, 0, 0)),
        ],
        out_specs=pl.BlockSpec((_B, 1, _D), lambda: (0, 0, 0)),
        out_shape=jax.ShapeDtypeStruct((_B, 1, _D), jnp.float32),
    )(prompt_lens, instruction_lens, dense_sums, instr_sums.reshape(_B, 1, _D))
    return out.reshape(_B, _D)
